# baseline (speedup 1.0000x reference)
import functools

import jax
import jax.numpy as jnp
from jax import lax
from jax.experimental import pallas as pl
from jax.experimental.pallas import tpu as pltpu
from jax.experimental.pallas import tpu_sc as plsc

NC = 2
NS = 16
NW = NC * NS
L = 16
CHUNK = 128
NBUF = 4
K = 2
NF = 26
FPAD = 32
DPAD = 128


def _fmt_body(idxT_hbm, out_hbm, tile_v, out_v):
    wid = lax.axis_index("s") * NC + lax.axis_index("c")
    B = idxT_hbm.shape[1]
    b_per_tile = B // NW
    n_blocks = b_per_tile // CHUNK
    lanes = lax.iota(jnp.int32, L)
    lane_pos = lanes * (2 * FPAD)
    zeros = jnp.zeros((L,), jnp.int32)

    def block(bb, carry):
        b0 = wid * b_per_tile + bb * CHUNK
        for fg in range(FPAD // 8):
            pltpu.sync_copy(
                idxT_hbm.at[pl.ds(fg * 8, 8), pl.ds(b0, CHUNK)], tile_v)
            for fr in range(8):
                f = fg * 8 + fr
                for gl in range(CHUNK // L):
                    dst = lane_pos + (gl * L * 2 * FPAD + 2 * f)
                    plsc.store_scatter(
                        out_v, [dst], tile_v[fr, pl.ds(gl * L, L)])
                    plsc.store_scatter(out_v, [dst + 1], zeros)
        pltpu.sync_copy(
            out_v, out_hbm.at[pl.ds(b0 * 2 * FPAD, CHUNK * 2 * FPAD)])
        return carry

    lax.fori_loop(0, n_blocks, block, 0)


def _emb_body(table_hbm, idx_hbm, out_hbm, idx_v, rows_v, gsem, wsem):
    wid = lax.axis_index("s") * NC + lax.axis_index("c")
    per_tile = idx_v.shape[0]
    n_chunks = per_tile // CHUNK
    n_outer = n_chunks // NBUF
    pos0 = wid * per_tile
    pltpu.sync_copy(idx_hbm.at[pl.ds(pos0, per_tile)], idx_v)

    def step(j, b, first_outer, last_outer):
        pltpu.make_async_copy(
            table_hbm.at[idx_v.at[pl.ds(j * CHUNK, CHUNK)]], rows_v.at[b],
            gsem.at[b]).wait()
        pltpu.async_copy(
            rows_v.at[b], out_hbm.at[pl.ds(pos0 + j * CHUNK, CHUNK)],
            wsem.at[b])
        if not (last_outer and b >= NBUF - K):
            b2 = (b + K) % NBUF
            if not (first_outer and b < NBUF - K):
                pltpu.make_async_copy(
                    rows_v.at[b2], out_hbm.at[pl.ds(0, CHUNK)],
                    wsem.at[b2]).wait()
            pltpu.async_copy(
                table_hbm.at[idx_v.at[pl.ds((j + K) * CHUNK, CHUNK)]],
                rows_v.at[b2], gsem.at[b2])

    for b in range(K):
        pltpu.async_copy(table_hbm.at[idx_v.at[pl.ds(b * CHUNK, CHUNK)]],
                         rows_v.at[b], gsem.at[b])

    for b in range(NBUF):
        step(b, b, True, False)

    def outer(g, carry):
        for b in range(NBUF):
            step(g * NBUF + b, b, False, False)
        return carry

    lax.fori_loop(1, n_outer - 1, outer, 0)

    for b in range(NBUF):
        step((n_outer - 1) * NBUF + b, b, False, True)

    for b in range(NBUF):
        pltpu.make_async_copy(
            rows_v.at[b], out_hbm.at[pl.ds(0, CHUNK)], wsem.at[b]).wait()


def kernel(input, weight):
    B, F = input.shape
    D = weight.shape[1]
    total_p = B * FPAD * 2

    mesh = plsc.VectorSubcoreMesh(core_axis_name="c", subcore_axis_name="s")

    fmt = functools.partial(
        pl.kernel,
        mesh=mesh,
        compiler_params=pltpu.CompilerParams(use_tc_tiling_on_sc=True,
                                             needs_layout_passes=False),
        out_type=jax.ShapeDtypeStruct((total_p,), jnp.int32),
        scratch_types=[
            pltpu.VMEM((8, CHUNK), jnp.int32),
            pltpu.VMEM((CHUNK * 2 * FPAD,), jnp.int32),
        ],
    )(_fmt_body)
    idx_flat = fmt(jnp.pad(input.T, ((0, FPAD - F), (0, 0))))

    emb = functools.partial(
        pl.kernel,
        mesh=mesh,
        compiler_params=pltpu.CompilerParams(use_tc_tiling_on_sc=False,
                                             needs_layout_passes=False),
        out_type=jax.ShapeDtypeStruct((total_p, D), weight.dtype),
        scratch_types=[
            pltpu.VMEM((total_p // NW,), jnp.int32),
            pltpu.VMEM((NBUF, CHUNK, D), jnp.float32),
            pltpu.SemaphoreType.DMA((NBUF,)),
            pltpu.SemaphoreType.DMA((NBUF,)),
        ],
    )(_emb_body)
    outp = emb(weight, idx_flat)
    return outp.reshape(B, FPAD, DPAD)[:, :F, :D]


# speedup vs baseline: 13.5344x; 13.5344x over previous
import functools

import jax
import jax.numpy as jnp
from jax import lax
from jax.experimental import pallas as pl
from jax.experimental.pallas import tpu as pltpu
from jax.experimental.pallas import tpu_sc as plsc

NC = 2
NS = 16
NW = NC * NS
CHUNK = 128
NBUF = 8
K = 4


def _emb_body(table_hbm, idx_hbm, out_hbm, idx_v, rows_v, gsem, wsem):
    wid = lax.axis_index("s") * NC + lax.axis_index("c")
    per_tile = idx_v.shape[0]
    n_chunks = per_tile // CHUNK
    n_outer = n_chunks // NBUF
    pltpu.sync_copy(idx_hbm.at[pl.ds(wid * per_tile, per_tile)], idx_v)

    def step(j, b, first_outer, last_outer):
        pltpu.make_async_copy(
            table_hbm.at[idx_v.at[pl.ds(j * CHUNK, CHUNK)]], rows_v.at[b],
            gsem.at[b]).wait()
        base = (wid * n_chunks + j) * CHUNK
        pltpu.async_copy(rows_v.at[b], out_hbm.at[pl.ds(base, CHUNK)],
                         wsem.at[b])
        if not (last_outer and b >= NBUF - K):
            b2 = (b + K) % NBUF
            if not (first_outer and b < NBUF - K):
                pltpu.make_async_copy(
                    rows_v.at[b2], out_hbm.at[pl.ds(0, CHUNK)],
                    wsem.at[b2]).wait()
            pltpu.async_copy(
                table_hbm.at[idx_v.at[pl.ds((j + K) * CHUNK, CHUNK)]],
                rows_v.at[b2], gsem.at[b2])

    for b in range(K):
        pltpu.async_copy(table_hbm.at[idx_v.at[pl.ds(b * CHUNK, CHUNK)]],
                         rows_v.at[b], gsem.at[b])

    for b in range(NBUF):
        step(b, b, True, False)

    def outer(g, carry):
        for b in range(NBUF):
            step(g * NBUF + b, b, False, False)
        return carry

    lax.fori_loop(1, n_outer - 1, outer, 0)

    for b in range(NBUF):
        step((n_outer - 1) * NBUF + b, b, False, True)

    for b in range(NBUF):
        pltpu.make_async_copy(
            rows_v.at[b], out_hbm.at[pl.ds(0, CHUNK)], wsem.at[b]).wait()


def kernel(input, weight):
    B, F = input.shape
    D = weight.shape[1]
    total = B * F
    idx = input.reshape(total)

    mesh = plsc.VectorSubcoreMesh(core_axis_name="c", subcore_axis_name="s")
    k = functools.partial(
        pl.kernel,
        mesh=mesh,
        compiler_params=pltpu.CompilerParams(use_tc_tiling_on_sc=False),
        out_type=jax.ShapeDtypeStruct((total, D), weight.dtype),
        scratch_types=[
            pltpu.VMEM((total // NW,), jnp.int32),
            pltpu.VMEM((NBUF, CHUNK, D), jnp.float32),
            pltpu.SemaphoreType.DMA((NBUF,)),
            pltpu.SemaphoreType.DMA((NBUF,)),
        ],
    )(_emb_body)
    out = k(weight, idx)
    return out.reshape(B, F, D)
